# baseline (speedup 1.0000x reference)
import functools

import jax
import jax.numpy as jnp
import numpy as np
from jax import lax
from jax.experimental import pallas as pl
from jax.experimental.pallas import tpu as pltpu
from jax.experimental.pallas import tpu_sc as plsc

VOCAB = 1000000
D = 64
BATCH = 4096
SEQ = 200

NC = 2
NS = 16
NW = NC * NS

BL = BATCH * SEQ
RPW = BL // NW
C = 200
NCHUNK = RPW // C
NBUF = 4


def _pe_table(max_len, d_emb):
    pos = np.arange(max_len, dtype=np.float64)[:, None]
    i = np.arange(d_emb, dtype=np.float64)[None, :]
    pe = pos / (10000.0 ** (2.0 * i / d_emb))
    pe[0, :] = 0.0
    pe[:, 0::2] = np.sin(pe[:, 0::2])
    pe[:, 1::2] = np.cos(pe[:, 1::2])
    return pe.astype(np.float32)


def _sc_embed(table, idx3, pe):
    mesh = plsc.VectorSubcoreMesh(core_axis_name="c", subcore_axis_name="s")

    @functools.partial(
        pl.kernel,
        out_type=jax.ShapeDtypeStruct((BL, D), jnp.float32),
        mesh=mesh,
        compiler_params=pltpu.CompilerParams(use_tc_tiling_on_sc=False),
        scratch_types=[
            pltpu.VMEM((NCHUNK, C), jnp.int32),
            pltpu.VMEM((C, D), jnp.float32),
            [pltpu.VMEM((C, D), jnp.float32) for _ in range(NBUF)],
            [pltpu.SemaphoreType.DMA for _ in range(NBUF)],
            [pltpu.SemaphoreType.DMA for _ in range(NBUF)],
        ],
    )
    def k(table_hbm, idx_hbm, pe_hbm, out_hbm, idx_v, pe_v, bufs, gsem, ssem):
        wid = lax.axis_index("s") * NC + lax.axis_index("c")
        base = wid * RPW

        pltpu.sync_copy(idx_hbm.at[wid], idx_v)
        pltpu.sync_copy(pe_hbm, pe_v)

        def start_gather(kc, b):
            pltpu.async_copy(table_hbm.at[idx_v.at[kc]], bufs[b], gsem[b])

        def wait_gather(kc, b):
            pltpu.make_async_copy(
                table_hbm.at[idx_v.at[kc]], bufs[b], gsem[b]).wait()

        def start_scatter(kc, b):
            pltpu.async_copy(
                bufs[b], out_hbm.at[pl.ds(base + kc * C, C)], ssem[b])

        def wait_scatter(kc, b):
            pltpu.make_async_copy(
                bufs[b], out_hbm.at[pl.ds(base + kc * C, C)], ssem[b]).wait()

        def add_pe(b):
            buf = bufs[b]

            @pl.loop(0, C)
            def _(l):
                for q in range(4):
                    s = pl.ds(q * 16, 16)
                    buf[l, s] = buf[l, s] + pe_v[l, s]

        start_gather(0, 0)
        start_gather(1, 1)

        for b in range(NBUF):
            wait_gather(b, b)
            add_pe(b)
            start_scatter(b, b)
            if b + 2 < NBUF:
                start_gather(b + 2, b + 2)
            else:
                b2 = (b + 2) % NBUF
                wait_scatter(b - 2, b2)
                start_gather(b + 2, b2)

        @pl.loop(NBUF, NCHUNK - NBUF, step=NBUF)
        def _(kk):
            for b in range(NBUF):
                kc = kk + b
                wait_gather(kc, b)
                add_pe(b)
                start_scatter(kc, b)
                b2 = (b + 2) % NBUF
                wait_scatter(kc - 2, b2)
                start_gather(kc + 2, b2)

        for b in range(NBUF):
            kc = NCHUNK - NBUF + b
            wait_gather(kc, b)
            add_pe(b)
            start_scatter(kc, b)
            b2 = (b + 2) % NBUF
            wait_scatter(kc - 2, b2)
            if kc + 2 < NCHUNK:
                start_gather(kc + 2, b2)

        wait_scatter(NCHUNK - 2, (NBUF - 2) % NBUF)
        wait_scatter(NCHUNK - 1, NBUF - 1)

    return k(table, idx3, pe)


_PE = _pe_table(SEQ, D)


def kernel(inputs, table):
    b, l = inputs.shape
    idx3 = inputs.reshape(NW, NCHUNK, C).astype(jnp.int32)
    pe = jnp.asarray(_PE)
    out = _sc_embed(table, idx3, pe)
    return out.reshape(b, l, D)
